# baseline (device time: 31329 ns/iter reference)
import jax
import jax.numpy as jnp
from jax import lax
from jax.experimental import pallas as pl
from jax.experimental.pallas import tpu as pltpu

N_DEV = 8
N_LAYERS = 3


def kernel(x, Win0, Wout0, Win1, Wout1, Win2, Wout2):
    b, d_sh = x.shape
    hdim = Win0.shape[1]

    def body(x_ref, win0_ref, wout0_ref, win1_ref, wout1_ref, win2_ref,
             wout2_ref, out_ref, send_buf, comm_ref, send_sems, recv_sems):
        my = lax.axis_index("i")
        wins = [win0_ref, win1_ref, win2_ref]
        wouts = [wout0_ref, wout1_ref, wout2_ref]

        all_rdmas = []
        x_cur = x_ref[...].astype(jnp.bfloat16)
        for l in range(N_LAYERS):
            partial = jnp.dot(
                x_cur, wins[l][...].astype(jnp.bfloat16),
                preferred_element_type=jnp.float32,
            ).astype(jnp.bfloat16)
            send_buf[l] = partial

            rdmas = []
            for k in range(1, N_DEV):
                peer = lax.rem(my + k, N_DEV)
                rdma = pltpu.make_async_remote_copy(
                    src_ref=send_buf.at[l],
                    dst_ref=comm_ref.at[l, k - 1],
                    send_sem=send_sems.at[l, k - 1],
                    recv_sem=recv_sems.at[l, k - 1],
                    device_id=(peer,),
                    device_id_type=pl.DeviceIdType.MESH,
                )
                rdma.start()
                rdmas.append(rdma)
            all_rdmas.extend(rdmas)

            acc = partial
            for k in range(1, N_DEV):
                rdmas[k - 1].wait_recv()
                acc = acc + comm_ref[l, k - 1]
            h = jnp.maximum(acc, 0)
            x_cur = jnp.dot(
                h, wouts[l][...].astype(jnp.bfloat16),
                preferred_element_type=jnp.float32,
            ).astype(jnp.bfloat16)

        out_ref[...] = x_cur.astype(jnp.float32)
        for rdma in all_rdmas:
            rdma.wait_send()

    return pl.pallas_call(
        body,
        out_shape=jax.ShapeDtypeStruct((b, d_sh), jnp.float32),
        in_specs=[pl.BlockSpec(memory_space=pltpu.VMEM)] * 7,
        out_specs=pl.BlockSpec(memory_space=pltpu.VMEM),
        scratch_shapes=[
            pltpu.VMEM((N_LAYERS, b, hdim), jnp.bfloat16),
            pltpu.VMEM((N_LAYERS, N_DEV - 1, b, hdim), jnp.bfloat16),
            pltpu.SemaphoreType.DMA((N_LAYERS, N_DEV - 1)),
            pltpu.SemaphoreType.DMA((N_LAYERS, N_DEV - 1)),
        ],
    )(x, Win0, Wout0, Win1, Wout1, Win2, Wout2)


# device time: 27020 ns/iter; 1.1595x vs baseline; 1.1595x over previous
import jax
import jax.numpy as jnp
from jax import lax
from jax.experimental import pallas as pl
from jax.experimental.pallas import tpu as pltpu

N_DEV = 8
N_LAYERS = 3


def kernel(x, Win0, Wout0, Win1, Wout1, Win2, Wout2):
    b, d_sh = x.shape
    hdim = Win0.shape[1]

    def body(x_ref, win0_ref, wout0_ref, win1_ref, wout1_ref, win2_ref,
             wout2_ref, out_ref, send_buf, comm_ref, send_sems, recv_sems):
        my = lax.axis_index("i")
        wins = [win0_ref, win1_ref, win2_ref]
        wouts = [wout0_ref, wout1_ref, wout2_ref]

        barrier_sem = pltpu.get_barrier_semaphore()
        pl.semaphore_signal(barrier_sem, inc=1, device_id=(my,),
                            device_id_type=pl.DeviceIdType.MESH)
        pl.semaphore_wait(barrier_sem, 1)

        all_rdmas = []
        x_cur = x_ref[...].astype(jnp.bfloat16)
        for l in range(N_LAYERS):
            partial = jnp.dot(
                x_cur, wins[l][...].astype(jnp.bfloat16),
                preferred_element_type=jnp.float32,
            ).astype(jnp.bfloat16)
            send_buf[l] = partial

            rdmas = []
            for k in range(1, N_DEV):
                peer = lax.rem(my + k, N_DEV)
                rdma = pltpu.make_async_remote_copy(
                    src_ref=send_buf.at[l],
                    dst_ref=comm_ref.at[l, k - 1],
                    send_sem=send_sems.at[l, k - 1],
                    recv_sem=recv_sems.at[l, k - 1],
                    device_id=(peer,),
                    device_id_type=pl.DeviceIdType.MESH,
                )
                rdma.start()
                rdmas.append(rdma)
            all_rdmas.extend(rdmas)

            acc = partial
            for k in range(1, N_DEV):
                rdmas[k - 1].wait_recv()
                acc = acc + comm_ref[l, k - 1]
            h = jnp.maximum(acc, 0)
            x_cur = jnp.dot(
                h, wouts[l][...].astype(jnp.bfloat16),
                preferred_element_type=jnp.float32,
            ).astype(jnp.bfloat16)

        out_ref[...] = x_cur.astype(jnp.float32)
        for rdma in all_rdmas:
            rdma.wait_send()

    return pl.pallas_call(
        body,
        out_shape=jax.ShapeDtypeStruct((b, d_sh), jnp.float32),
        in_specs=[pl.BlockSpec(memory_space=pltpu.VMEM)] * 7,
        out_specs=pl.BlockSpec(memory_space=pltpu.VMEM),
        scratch_shapes=[
            pltpu.VMEM((N_LAYERS, b, hdim), jnp.bfloat16),
            pltpu.VMEM((N_LAYERS, N_DEV - 1, b, hdim), jnp.bfloat16),
            pltpu.SemaphoreType.DMA((N_LAYERS, N_DEV - 1)),
            pltpu.SemaphoreType.DMA((N_LAYERS, N_DEV - 1)),
        ],
        compiler_params=pltpu.CompilerParams(collective_id=0),
    )(x, Win0, Wout0, Win1, Wout1, Win2, Wout2)
